# baseline (speedup 1.0000x reference)
import functools

import jax
import jax.numpy as jnp
from jax.experimental import pallas as pl
from jax.experimental.pallas import tpu as pltpu
from jax.experimental.pallas import tpu_sc as plsc

_GCN_K = 16
_PPF_K = 32
_R2 = 0.3 ** 2
_EPS = 1e-5


def _leaky(x):
    return jnp.where(x >= 0.0, x, 0.2 * x)


def _knn_bq_body(crow_ref, call_ref, knn_ref, bq_ref,
                 sq_ref, hit_ref, kn_s, bq_s,
                 *, n_tot, nc, cw, rows):
    f32 = jnp.float32
    i32 = jnp.int32
    bigi = i32(nc * cw + 1)
    ax = crow_ref[:, 0:1]
    ay = crow_ref[:, 1:2]
    az = crow_ref[:, 2:3]
    aa = ax * ax + ay * ay + az * az
    rx = ax.astype(jnp.bfloat16).astype(jnp.float32)
    ry = ay.astype(jnp.bfloat16).astype(jnp.float32)
    rz = az.astype(jnp.bfloat16).astype(jnp.float32)

    nf = cw // 128

    def _fold(x, acc):
        return jnp.minimum(acc, jnp.min(x.reshape(rows, nf, 128), axis=1))

    def _lanemin(acc):
        return jnp.min(acc, axis=-1, keepdims=True)

    def fill_body(j, macc):
        bx = call_ref[j, 0:1, :]
        by = call_ref[j, 1:2, :]
        bz = call_ref[j, 2:3, :]
        sx = bx.astype(jnp.bfloat16).astype(jnp.float32)
        sy = by.astype(jnp.bfloat16).astype(jnp.float32)
        sz = bz.astype(jnp.bfloat16).astype(jnp.float32)
        ab = rx * sx + ry * sy + rz * sz
        bb = bx * bx + by * by + bz * bz
        sq = aa - 2.0 * ab + bb
        sq_ref[j] = sq
        col = j * cw + jax.lax.broadcasted_iota(i32, (rows, cw), 1)
        hit_ref[j] = jnp.where(sq < f32(_R2), col, n_tot)
        return _fold(sq, macc)

    m0 = _lanemin(jax.lax.fori_loop(0, nc, fill_body,
                                    jnp.full((rows, 128), jnp.inf, f32)))

    def bq_body(t, lastc):
        def mn(j, acc):
            h = hit_ref[j]
            return _fold(jnp.where(h > lastc, h, n_tot), acc)
        m = _lanemin(jax.lax.fori_loop(0, nc, mn,
                                       jnp.full((rows, 128), n_tot, i32)))
        bq_s[t] = m
        return m

    jax.lax.fori_loop(0, _PPF_K, bq_body,
                      jnp.full((rows, 1), -1, i32))

    def ix_pass(m):
        def ix(j, acc):
            ch = sq_ref[j]
            col = j * cw + jax.lax.broadcasted_iota(i32, (rows, cw), 1)
            return _fold(jnp.where(ch == m, col, bigi), acc)
        return _lanemin(jax.lax.fori_loop(0, nc, ix,
                                          jnp.full((rows, 128), bigi, i32)))

    def kn_body(t, m):
        ii = ix_pass(m)
        kn_s[t] = ii

        def mskmin(j, acc):
            col = j * cw + jax.lax.broadcasted_iota(i32, (rows, cw), 1)
            v = jnp.where(col == ii, jnp.inf, sq_ref[j])
            sq_ref[j] = v
            return _fold(v, acc)

        return _lanemin(jax.lax.fori_loop(0, nc, mskmin,
                                          jnp.full((rows, 128), jnp.inf,
                                                   f32)))

    m_last = jax.lax.fori_loop(0, _GCN_K, kn_body, m0)
    kn_s[_GCN_K] = ix_pass(m_last)

    first = bq_s[0]
    for s in range(_PPF_K):
        v = bq_s[s]
        bq_ref[:, s:s + 1] = jnp.where(v == n_tot, first, v)
    for t in range(1, _GCN_K + 1):
        knn_ref[:, t - 1:t] = kn_s[t]


def _knn_ballquery(c_pad8, call3, *, n_tot, rows, nc, cw):
    nblk = n_tot // rows
    body = functools.partial(_knn_bq_body, n_tot=n_tot, nc=nc, cw=cw,
                             rows=rows)
    return pl.pallas_call(
        body,
        grid=(nblk,),
        in_specs=[
            pl.BlockSpec((rows, 8), lambda i: (i, 0)),
            pl.BlockSpec((nc, 8, cw), lambda i: (0, 0, 0)),
        ],
        out_specs=[
            pl.BlockSpec((rows, _GCN_K), lambda i: (i, 0)),
            pl.BlockSpec((rows, _PPF_K), lambda i: (i, 0)),
        ],
        out_shape=[
            jax.ShapeDtypeStruct((n_tot, _GCN_K), jnp.int32),
            jax.ShapeDtypeStruct((n_tot, _PPF_K), jnp.int32),
        ],
        scratch_shapes=[
            pltpu.VMEM((nc, rows, cw), jnp.float32),
            pltpu.VMEM((nc, rows, cw), jnp.int32),
            pltpu.VMEM((_GCN_K + 1, rows, 1), jnp.int32),
            pltpu.VMEM((_PPF_K, rows, 1), jnp.int32),
        ],
    )(c_pad8, call3)


def _sc_gather(data, idx_flat, window):
    ni = idx_flat.shape[0]
    d = data.shape[1]
    assert ni % window == 0
    mesh = plsc.VectorSubcoreMesh(core_axis_name="c", subcore_axis_name="s")
    idx2 = idx_flat.reshape(1, ni)

    @functools.partial(
        pl.kernel,
        out_type=jax.ShapeDtypeStruct((ni, d), data.dtype),
        mesh=mesh,
    )
    def gather_kernel(x_hbm, i_hbm, o_hbm):
        def body(i_vmem, o_vmem):
            pltpu.sync_copy(x_hbm.at[i_vmem.at[0]], o_vmem)

        pltpu.emit_pipeline(
            body,
            grid=(ni // window,),
            in_specs=[pl.BlockSpec((1, window), index_map=lambda i: (0, i))],
            out_specs=[pl.BlockSpec((window, d), index_map=lambda i: (i, 0))],
            core_axis_name=("c", "s"),
            dimension_semantics=(pltpu.PARALLEL,),
        )(i_hbm, o_hbm)

    return gather_kernel(data, idx2)


def _nbr_reduce_body(qg_ref, p_ref, m_ref, s_ref):
    qg = qg_ref[...]
    mx = jnp.max(qg, axis=1)
    sm = jnp.sum(qg, axis=1)
    s2 = jnp.sum(qg * qg, axis=1)
    m_ref[...] = mx
    p = p_ref[...]

    @pl.when(pl.program_id(0) == 0)
    def _():
        s_ref[...] = jnp.zeros_like(s_ref)

    s_ref[0:1, :] += jnp.sum(p, axis=0, keepdims=True)
    s_ref[1:2, :] += jnp.sum(p * p, axis=0, keepdims=True)
    s_ref[2:3, :] += jnp.sum(p * sm, axis=0, keepdims=True)
    s_ref[3:4, :] += jnp.sum(sm, axis=0, keepdims=True)
    s_ref[4:5, :] += jnp.sum(s2, axis=0, keepdims=True)


def _nbr_reduce2_body(qa_ref, qb_ref, p_ref, m_ref, s_ref):
    c = qa_ref.shape[2]
    p = p_ref[...]

    @pl.when(pl.program_id(0) == 0)
    def _():
        s_ref[...] = jnp.zeros_like(s_ref)

    for h, qref in ((0, qa_ref), (1, qb_ref)):
        lo, hi = h * c, (h + 1) * c
        qg = qref[...]
        mx = jnp.max(qg, axis=1)
        sm = jnp.sum(qg, axis=1)
        s2 = jnp.sum(qg * qg, axis=1)
        m_ref[:, lo:hi] = mx
        ph = p[:, lo:hi]
        s_ref[0:1, lo:hi] += jnp.sum(ph, axis=0, keepdims=True)
        s_ref[1:2, lo:hi] += jnp.sum(ph * ph, axis=0, keepdims=True)
        s_ref[2:3, lo:hi] += jnp.sum(ph * sm, axis=0, keepdims=True)
        s_ref[3:4, lo:hi] += jnp.sum(sm, axis=0, keepdims=True)
        s_ref[4:5, lo:hi] += jnp.sum(s2, axis=0, keepdims=True)


def _nbr_reduce2(qa, qb, p, *, rows):
    n, k, c = qa.shape
    return pl.pallas_call(
        _nbr_reduce2_body,
        grid=(n // rows,),
        in_specs=[
            pl.BlockSpec((rows, k, c), lambda i: (i, 0, 0)),
            pl.BlockSpec((rows, k, c), lambda i: (i, 0, 0)),
            pl.BlockSpec((rows, 2 * c), lambda i: (i, 0)),
        ],
        out_specs=[
            pl.BlockSpec((rows, 2 * c), lambda i: (i, 0)),
            pl.BlockSpec((8, 2 * c), lambda i: (0, 0)),
        ],
        out_shape=[
            jax.ShapeDtypeStruct((n, 2 * c), jnp.float32),
            jax.ShapeDtypeStruct((8, 2 * c), jnp.float32),
        ],
    )(qa, qb, p)


def _nbr_reduce(qg, p, *, rows):
    n, k, c = qg.shape
    return pl.pallas_call(
        _nbr_reduce_body,
        grid=(n // rows,),
        in_specs=[
            pl.BlockSpec((rows, k, c), lambda i: (i, 0, 0)),
            pl.BlockSpec((rows, c), lambda i: (i, 0)),
        ],
        out_specs=[
            pl.BlockSpec((rows, c), lambda i: (i, 0)),
            pl.BlockSpec((8, c), lambda i: (0, 0)),
        ],
        out_shape=[
            jax.ShapeDtypeStruct((n, c), jnp.float32),
            jax.ShapeDtypeStruct((8, c), jnp.float32),
        ],
    )(qg, p)


def _edge_norm_consts(stats, n, k):
    cnt = jnp.float32(n * k)
    sp, sp2, sps, sq, sq2 = (stats[0], stats[1], stats[2], stats[3], stats[4])
    mean = (k * sp + sq) / cnt
    e2 = (k * sp2 + 2.0 * sps + sq2) / cnt
    var = e2 - mean * mean
    inv = jax.lax.rsqrt(var + _EPS)
    z = jnp.zeros_like(mean)
    return jnp.stack([mean, inv, z, z, z, z, z, z], axis=0)


def _plain_norm_consts(stats, n):
    cnt = jnp.float32(n)
    mean = stats[0] / cnt
    var = stats[1] / cnt - mean * mean
    inv = jax.lax.rsqrt(var + _EPS)
    z = jnp.zeros_like(mean)
    return jnp.stack([mean, inv, z, z, z, z, z, z], axis=0)


def _ppf_body(gc_ref, c_ref, n_ref, p_ref):
    gx = gc_ref[:, :, 0] - c_ref[:, 0:1]
    gy = gc_ref[:, :, 1] - c_ref[:, 1:2]
    gz = gc_ref[:, :, 2] - c_ref[:, 2:3]
    nx = n_ref[:, 0:1]
    ny = n_ref[:, 1:2]
    nz = n_ref[:, 2:3]
    cx = ny * gz - nz * gy
    cy = nz * gx - nx * gz
    cz = nx * gy - ny * gx
    ss = cx * cx + cy * cy + cz * cz
    pos = (ss > 0.0).astype(jnp.float32)
    nr = jnp.sqrt(jnp.where(ss > 0.0, ss, 1.0)) * pos
    dt = nx * gx + ny * gy + nz * gz
    bz = jnp.logical_and(ss == 0.0, dt == 0.0)
    ang = jnp.arctan2(nr, jnp.where(bz, 1.0, dt))
    amax = jnp.max(ang, axis=1, keepdims=True)
    p_ref[...] = jnp.broadcast_to(amax, p_ref.shape)


def _ppf(gc3, c_pad, n_pad, *, rows):
    n = c_pad.shape[0]
    return pl.pallas_call(
        _ppf_body,
        grid=(n // rows,),
        in_specs=[
            pl.BlockSpec((rows, _PPF_K, 128), lambda i: (i, 0, 0)),
            pl.BlockSpec((rows, 16), lambda i: (i, 0)),
            pl.BlockSpec((rows, 16), lambda i: (i, 0)),
        ],
        out_specs=pl.BlockSpec((rows, 128), lambda i: (i, 0)),
        out_shape=jax.ShapeDtypeStruct((n, 128), jnp.float32),
    )(gc3, c_pad, n_pad)


def _mm2_body(x_ref, wa_ref, wb_ref, oa_ref, ob_ref):
    x = x_ref[...]
    oa_ref[...] = jnp.dot(x, wa_ref[...], preferred_element_type=jnp.float32)
    ob_ref[...] = jnp.dot(x, wb_ref[...], preferred_element_type=jnp.float32)


def _mm2(x, wa, wb, *, rows):
    n, c = x.shape
    ca = wa.shape[1]
    cb = wb.shape[1]
    return pl.pallas_call(
        _mm2_body,
        grid=(n // rows,),
        in_specs=[
            pl.BlockSpec((rows, c), lambda i: (i, 0)),
            pl.BlockSpec(wa.shape, lambda i: (0, 0)),
            pl.BlockSpec(wb.shape, lambda i: (0, 0)),
        ],
        out_specs=[
            pl.BlockSpec((rows, ca), lambda i: (i, 0)),
            pl.BlockSpec((rows, cb), lambda i: (i, 0)),
        ],
        out_shape=[
            jax.ShapeDtypeStruct((n, ca), jnp.float32),
            jax.ShapeDtypeStruct((n, cb), jnp.float32),
        ],
    )(x, wa, wb)


def _edge_mm_body(p_ref, m_ref, st_ref, wa_ref, wb_ref, f_ref, oa_ref,
                  ob_ref):
    mean = st_ref[0:1, :]
    inv = st_ref[1:2, :]
    f = _leaky((p_ref[...] + m_ref[...] - mean) * inv)
    f_ref[...] = f
    oa_ref[...] = jnp.dot(f, wa_ref[...], preferred_element_type=jnp.float32)
    ob_ref[...] = jnp.dot(f, wb_ref[...], preferred_element_type=jnp.float32)


def _edge_mm(p, m, st, wa, wb, *, rows):
    n, c = p.shape
    ca = wa.shape[1]
    cb = wb.shape[1]
    return pl.pallas_call(
        _edge_mm_body,
        grid=(n // rows,),
        in_specs=[
            pl.BlockSpec((rows, c), lambda i: (i, 0)),
            pl.BlockSpec((rows, c), lambda i: (i, 0)),
            pl.BlockSpec((8, c), lambda i: (0, 0)),
            pl.BlockSpec(wa.shape, lambda i: (0, 0)),
            pl.BlockSpec(wb.shape, lambda i: (0, 0)),
        ],
        out_specs=[
            pl.BlockSpec((rows, c), lambda i: (i, 0)),
            pl.BlockSpec((rows, ca), lambda i: (i, 0)),
            pl.BlockSpec((rows, cb), lambda i: (i, 0)),
        ],
        out_shape=[
            jax.ShapeDtypeStruct((n, c), jnp.float32),
            jax.ShapeDtypeStruct((n, ca), jnp.float32),
            jax.ShapeDtypeStruct((n, cb), jnp.float32),
        ],
    )(p, m, st, wa, wb)


def _stats_accum(s_ref, h, pid):
    @pl.when(pid == 0)
    def _():
        s_ref[...] = jnp.zeros_like(s_ref)

    s_ref[0:1, :] += jnp.sum(h, axis=0, keepdims=True)
    s_ref[1:2, :] += jnp.sum(h * h, axis=0, keepdims=True)


def _conv3_body(p2_ref, m2_ref, st2_ref, x_ref, f1_ref, w3a_ref, w3b_ref,
                w3c_ref, h3_ref, s_ref):
    mean = st2_ref[0:1, :]
    inv = st2_ref[1:2, :]
    f2 = _leaky((p2_ref[...] + m2_ref[...] - mean) * inv)
    h3 = (jnp.dot(x_ref[...], w3a_ref[...], preferred_element_type=jnp.float32)
          + jnp.dot(f1_ref[...], w3b_ref[...],
                    preferred_element_type=jnp.float32)
          + jnp.dot(f2, w3c_ref[...], preferred_element_type=jnp.float32))
    h3_ref[...] = h3
    _stats_accum(s_ref, h3, pl.program_id(0))


def _conv3(p2, m2, st2, x, f1, w3a, w3b, w3c, *, rows):
    n, c2 = p2.shape
    c = x.shape[1]
    co = w3a.shape[1]
    return pl.pallas_call(
        _conv3_body,
        grid=(n // rows,),
        in_specs=[
            pl.BlockSpec((rows, c2), lambda i: (i, 0)),
            pl.BlockSpec((rows, c2), lambda i: (i, 0)),
            pl.BlockSpec((8, c2), lambda i: (0, 0)),
            pl.BlockSpec((rows, c), lambda i: (i, 0)),
            pl.BlockSpec((rows, c), lambda i: (i, 0)),
            pl.BlockSpec(w3a.shape, lambda i: (0, 0)),
            pl.BlockSpec(w3b.shape, lambda i: (0, 0)),
            pl.BlockSpec(w3c.shape, lambda i: (0, 0)),
        ],
        out_specs=[
            pl.BlockSpec((rows, co), lambda i: (i, 0)),
            pl.BlockSpec((8, co), lambda i: (0, 0)),
        ],
        out_shape=[
            jax.ShapeDtypeStruct((n, co), jnp.float32),
            jax.ShapeDtypeStruct((8, co), jnp.float32),
        ],
    )(p2, m2, st2, x, f1, w3a, w3b, w3c)


def _fc1_body(h3_ref, st_ref, p_ref, w_ref, wp_ref, b_ref, h4_ref, s_ref):
    mean = st_ref[0:1, :]
    inv = st_ref[1:2, :]
    g = _leaky((h3_ref[...] - mean) * inv)
    h4 = (jnp.dot(g, w_ref[...], preferred_element_type=jnp.float32)
          + p_ref[:, 0:1] * wp_ref[0:1, :] + b_ref[0:1, :])
    h4_ref[...] = h4
    _stats_accum(s_ref, h4, pl.program_id(0))


def _fc1(h3, st, p, w, wp, b, *, rows):
    n, c = h3.shape
    co = w.shape[1]
    return pl.pallas_call(
        _fc1_body,
        grid=(n // rows,),
        in_specs=[
            pl.BlockSpec((rows, c), lambda i: (i, 0)),
            pl.BlockSpec((8, c), lambda i: (0, 0)),
            pl.BlockSpec((rows, 128), lambda i: (i, 0)),
            pl.BlockSpec(w.shape, lambda i: (0, 0)),
            pl.BlockSpec(wp.shape, lambda i: (0, 0)),
            pl.BlockSpec(b.shape, lambda i: (0, 0)),
        ],
        out_specs=[
            pl.BlockSpec((rows, co), lambda i: (i, 0)),
            pl.BlockSpec((8, co), lambda i: (0, 0)),
        ],
        out_shape=[
            jax.ShapeDtypeStruct((n, co), jnp.float32),
            jax.ShapeDtypeStruct((8, co), jnp.float32),
        ],
    )(h3, st, p, w, wp, b)


def _fc2_body(h4_ref, st_ref, w_ref, b_ref, h5_ref, s_ref):
    mean = st_ref[0:1, :]
    inv = st_ref[1:2, :]
    g = _leaky((h4_ref[...] - mean) * inv)
    h5 = jnp.dot(g, w_ref[...], preferred_element_type=jnp.float32) + b_ref[0:1, :]
    h5_ref[...] = h5
    _stats_accum(s_ref, h5, pl.program_id(0))


def _fc2(h4, st, w, b, *, rows):
    n, c = h4.shape
    co = w.shape[1]
    return pl.pallas_call(
        _fc2_body,
        grid=(n // rows,),
        in_specs=[
            pl.BlockSpec((rows, c), lambda i: (i, 0)),
            pl.BlockSpec((8, c), lambda i: (0, 0)),
            pl.BlockSpec(w.shape, lambda i: (0, 0)),
            pl.BlockSpec(b.shape, lambda i: (0, 0)),
        ],
        out_specs=[
            pl.BlockSpec((rows, co), lambda i: (i, 0)),
            pl.BlockSpec((8, co), lambda i: (0, 0)),
        ],
        out_shape=[
            jax.ShapeDtypeStruct((n, co), jnp.float32),
            jax.ShapeDtypeStruct((8, co), jnp.float32),
        ],
    )(h4, st, w, b)


def _final_body(h5_ref, st_ref, o_ref):
    mean = st_ref[0:1, :]
    inv = st_ref[1:2, :]
    o_ref[...] = _leaky((h5_ref[...] - mean) * inv)


def _final(h5, st, *, rows):
    n, c = h5.shape
    return pl.pallas_call(
        _final_body,
        grid=(n // rows,),
        in_specs=[
            pl.BlockSpec((rows, c), lambda i: (i, 0)),
            pl.BlockSpec((8, c), lambda i: (0, 0)),
        ],
        out_specs=pl.BlockSpec((rows, c), lambda i: (i, 0)),
        out_shape=jax.ShapeDtypeStruct((n, c), jnp.float32),
    )(h5, st)


def kernel(coords, feats, normals, W1, W2, W3, Wf1, bf1, Wf2, bf2):
    b, _, n = coords.shape
    c = feats.shape[1]
    assert b == 1
    rows = 200 if n % 200 == 0 else 8
    cw = 512 if n > 512 else 128
    nc = -(-n // cw)
    npad = nc * cw

    cT = coords[0].T
    fT = feats[0].T
    nT = normals[0].T

    c_pad8 = jnp.pad(cT, ((0, 0), (0, 5)))
    c_pad16 = jnp.pad(cT, ((0, 0), (0, 13)))
    c_pad128 = jnp.pad(cT, ((0, 0), (0, 125)))
    n_pad16 = jnp.pad(nT, ((0, 0), (0, 13)))
    call = jnp.pad(cT.T, ((0, 0), (0, npad - n)), constant_values=1e6)
    call = jnp.pad(call, ((0, 5), (0, 0)))
    call3 = call.reshape(8, nc, cw).transpose(1, 0, 2)

    knn, bq = _knn_ballquery(c_pad8, call3, n_tot=n, rows=rows, nc=nc, cw=cw)

    gc = _sc_gather(c_pad128, bq.reshape(n * _PPF_K), 128)
    p = _ppf(gc.reshape(n, _PPF_K, 128), c_pad16, n_pad16, rows=rows)

    w1d = (W1[:, :c] - W1[:, c:]).T
    w1b = W1[:, c:].T
    p1, q1 = _mm2(fT, w1d, w1b, rows=rows)
    qg1 = _sc_gather(q1, knn.reshape(n * _GCN_K), 128)
    m1, st1 = _nbr_reduce(qg1.reshape(n, _GCN_K, c), p1, rows=rows)
    ncs1 = _edge_norm_consts(st1, n, _GCN_K)

    w2d = (W2[:, :c] - W2[:, c:]).T
    w2b = W2[:, c:].T
    f1, p2, q2 = _edge_mm(p1, m1, ncs1, w2d, w2b, rows=rows)
    knn_flat = knn.reshape(n * _GCN_K)
    qg2a = _sc_gather(q2[:, :c], knn_flat, 128)
    qg2b = _sc_gather(q2[:, c:], knn_flat, 128)
    m2, st2 = _nbr_reduce2(qg2a.reshape(n, _GCN_K, c),
                           qg2b.reshape(n, _GCN_K, c), p2, rows=rows)
    ncs2 = _edge_norm_consts(st2, n, _GCN_K)

    w3a = W3[:, :c].T
    w3b = W3[:, c:2 * c].T
    w3c = W3[:, 2 * c:].T
    h3, st3 = _conv3(p2, m2, ncs2, fT, f1, w3a, w3b, w3c, rows=rows)
    ncs3 = _plain_norm_consts(st3, n)

    wp = Wf1[:, 0:1].T
    wf1g = Wf1[:, 1:].T
    bf1r = bf1.reshape(1, -1)
    h4, st4 = _fc1(h3, ncs3, p, wf1g, wp, bf1r, rows=rows)
    ncs4 = _plain_norm_consts(st4, n)

    h5, st5 = _fc2(h4, ncs4, Wf2.T, bf2.reshape(1, -1), rows=rows)
    ncs5 = _plain_norm_consts(st5, n)

    out = _final(h5, ncs5, rows=rows)
    return out.T[None]


# speedup vs baseline: 3.9156x; 3.9156x over previous
import functools

import jax
import jax.numpy as jnp
from jax.experimental import pallas as pl
from jax.experimental.pallas import tpu as pltpu
from jax.experimental.pallas import tpu_sc as plsc

_GCN_K = 16
_PPF_K = 32
_R2 = 0.3 ** 2
_EPS = 1e-5


def _leaky(x):
    return jnp.where(x >= 0.0, x, 0.2 * x)


def _knn_bq_body(crow_ref, call_ref, knn_ref, bq_ref,
                 sq_ref, hit_ref, kn_s, bq_s,
                 *, n_tot, nc, cw, rows):
    f32 = jnp.float32
    i32 = jnp.int32
    bigi = i32(nc * cw + 1)
    ax = crow_ref[:, 0:1]
    ay = crow_ref[:, 1:2]
    az = crow_ref[:, 2:3]
    aa = ax * ax + ay * ay + az * az
    rx = ax.astype(jnp.bfloat16).astype(jnp.float32)
    ry = ay.astype(jnp.bfloat16).astype(jnp.float32)
    rz = az.astype(jnp.bfloat16).astype(jnp.float32)

    def _fold(x, acc):
        return jnp.minimum(acc, x)

    def _lanemin(acc):
        return jnp.min(acc, axis=-1, keepdims=True)

    def fill_body(j, macc):
        bx = call_ref[j, 0:1, :]
        by = call_ref[j, 1:2, :]
        bz = call_ref[j, 2:3, :]
        sx = bx.astype(jnp.bfloat16).astype(jnp.float32)
        sy = by.astype(jnp.bfloat16).astype(jnp.float32)
        sz = bz.astype(jnp.bfloat16).astype(jnp.float32)
        ab = rx * sx + ry * sy + rz * sz
        bb = bx * bx + by * by + bz * bz
        sq = aa - 2.0 * ab + bb
        sq_ref[j] = sq
        col = j * cw + jax.lax.broadcasted_iota(i32, (rows, cw), 1)
        hit_ref[j] = jnp.where(sq < f32(_R2), col, n_tot)
        return _fold(sq, macc)

    m0 = _lanemin(jax.lax.fori_loop(0, nc, fill_body,
                                    jnp.full((rows, cw), jnp.inf, f32)))

    def bq_body(t, lastc):
        def mn(j, acc):
            h = hit_ref[j]
            return _fold(jnp.where(h > lastc, h, n_tot), acc)
        m = _lanemin(jax.lax.fori_loop(0, nc, mn,
                                       jnp.full((rows, cw), n_tot, i32)))
        bq_s[t] = m
        return m

    jax.lax.fori_loop(0, _PPF_K, bq_body,
                      jnp.full((rows, 1), -1, i32))

    def ix_pass(m):
        def ix(j, acc):
            ch = sq_ref[j]
            col = j * cw + jax.lax.broadcasted_iota(i32, (rows, cw), 1)
            return _fold(jnp.where(ch == m, col, bigi), acc)
        return _lanemin(jax.lax.fori_loop(0, nc, ix,
                                          jnp.full((rows, cw), bigi, i32)))

    def kn_body(t, m):
        ii = ix_pass(m)
        kn_s[t] = ii

        def mskmin(j, acc):
            col = j * cw + jax.lax.broadcasted_iota(i32, (rows, cw), 1)
            v = jnp.where(col == ii, jnp.inf, sq_ref[j])
            sq_ref[j] = v
            return _fold(v, acc)

        return _lanemin(jax.lax.fori_loop(0, nc, mskmin,
                                          jnp.full((rows, cw), jnp.inf, f32)))

    m_last = jax.lax.fori_loop(0, _GCN_K, kn_body, m0)
    kn_s[_GCN_K] = ix_pass(m_last)

    first = bq_s[0]
    for s in range(_PPF_K):
        v = bq_s[s]
        bq_ref[:, s:s + 1] = jnp.where(v == n_tot, first, v)
    for t in range(1, _GCN_K + 1):
        knn_ref[:, t - 1:t] = kn_s[t]


def _knn_ballquery(c_pad8, call3, *, n_tot, rows, nc, cw):
    nblk = n_tot // rows
    body = functools.partial(_knn_bq_body, n_tot=n_tot, nc=nc, cw=cw,
                             rows=rows)
    return pl.pallas_call(
        body,
        grid=(nblk,),
        in_specs=[
            pl.BlockSpec((rows, 8), lambda i: (i, 0)),
            pl.BlockSpec((nc, 8, cw), lambda i: (0, 0, 0)),
        ],
        out_specs=[
            pl.BlockSpec((rows, _GCN_K), lambda i: (i, 0)),
            pl.BlockSpec((rows, _PPF_K), lambda i: (i, 0)),
        ],
        out_shape=[
            jax.ShapeDtypeStruct((n_tot, _GCN_K), jnp.int32),
            jax.ShapeDtypeStruct((n_tot, _PPF_K), jnp.int32),
        ],
        scratch_shapes=[
            pltpu.VMEM((nc, rows, cw), jnp.float32),
            pltpu.VMEM((nc, rows, cw), jnp.int32),
            pltpu.VMEM((_GCN_K + 1, rows, 1), jnp.int32),
            pltpu.VMEM((_PPF_K, rows, 1), jnp.int32),
        ],
    )(c_pad8, call3)


def _sc_gather(data, idx_flat, window):
    ni = idx_flat.shape[0]
    d = data.shape[1]
    assert ni % window == 0
    mesh = plsc.VectorSubcoreMesh(core_axis_name="c", subcore_axis_name="s")
    idx2 = idx_flat.reshape(1, ni)

    @functools.partial(
        pl.kernel,
        out_type=jax.ShapeDtypeStruct((ni, d), data.dtype),
        mesh=mesh,
    )
    def gather_kernel(x_hbm, i_hbm, o_hbm):
        def body(i_vmem, o_vmem):
            pltpu.sync_copy(x_hbm.at[i_vmem.at[0]], o_vmem)

        pltpu.emit_pipeline(
            body,
            grid=(ni // window,),
            in_specs=[pl.BlockSpec((1, window), index_map=lambda i: (0, i))],
            out_specs=[pl.BlockSpec((window, d), index_map=lambda i: (i, 0))],
            core_axis_name=("c", "s"),
            dimension_semantics=(pltpu.PARALLEL,),
        )(i_hbm, o_hbm)

    return gather_kernel(data, idx2)


def _nbr_reduce_body(qg_ref, p_ref, m_ref, s_ref):
    qg = qg_ref[...]
    mx = jnp.max(qg, axis=1)
    sm = jnp.sum(qg, axis=1)
    s2 = jnp.sum(qg * qg, axis=1)
    m_ref[...] = mx
    p = p_ref[...]

    @pl.when(pl.program_id(0) == 0)
    def _():
        s_ref[...] = jnp.zeros_like(s_ref)

    s_ref[0:1, :] += jnp.sum(p, axis=0, keepdims=True)
    s_ref[1:2, :] += jnp.sum(p * p, axis=0, keepdims=True)
    s_ref[2:3, :] += jnp.sum(p * sm, axis=0, keepdims=True)
    s_ref[3:4, :] += jnp.sum(sm, axis=0, keepdims=True)
    s_ref[4:5, :] += jnp.sum(s2, axis=0, keepdims=True)


def _nbr_reduce2_body(qa_ref, qb_ref, p_ref, m_ref, s_ref):
    c = qa_ref.shape[2]
    p = p_ref[...]

    @pl.when(pl.program_id(0) == 0)
    def _():
        s_ref[...] = jnp.zeros_like(s_ref)

    for h, qref in ((0, qa_ref), (1, qb_ref)):
        lo, hi = h * c, (h + 1) * c
        qg = qref[...]
        mx = jnp.max(qg, axis=1)
        sm = jnp.sum(qg, axis=1)
        s2 = jnp.sum(qg * qg, axis=1)
        m_ref[:, lo:hi] = mx
        ph = p[:, lo:hi]
        s_ref[0:1, lo:hi] += jnp.sum(ph, axis=0, keepdims=True)
        s_ref[1:2, lo:hi] += jnp.sum(ph * ph, axis=0, keepdims=True)
        s_ref[2:3, lo:hi] += jnp.sum(ph * sm, axis=0, keepdims=True)
        s_ref[3:4, lo:hi] += jnp.sum(sm, axis=0, keepdims=True)
        s_ref[4:5, lo:hi] += jnp.sum(s2, axis=0, keepdims=True)


def _nbr_reduce2(qa, qb, p, *, rows):
    n, k, c = qa.shape
    return pl.pallas_call(
        _nbr_reduce2_body,
        grid=(n // rows,),
        in_specs=[
            pl.BlockSpec((rows, k, c), lambda i: (i, 0, 0)),
            pl.BlockSpec((rows, k, c), lambda i: (i, 0, 0)),
            pl.BlockSpec((rows, 2 * c), lambda i: (i, 0)),
        ],
        out_specs=[
            pl.BlockSpec((rows, 2 * c), lambda i: (i, 0)),
            pl.BlockSpec((8, 2 * c), lambda i: (0, 0)),
        ],
        out_shape=[
            jax.ShapeDtypeStruct((n, 2 * c), jnp.float32),
            jax.ShapeDtypeStruct((8, 2 * c), jnp.float32),
        ],
    )(qa, qb, p)


def _nbr_reduce(qg, p, *, rows):
    n, k, c = qg.shape
    return pl.pallas_call(
        _nbr_reduce_body,
        grid=(n // rows,),
        in_specs=[
            pl.BlockSpec((rows, k, c), lambda i: (i, 0, 0)),
            pl.BlockSpec((rows, c), lambda i: (i, 0)),
        ],
        out_specs=[
            pl.BlockSpec((rows, c), lambda i: (i, 0)),
            pl.BlockSpec((8, c), lambda i: (0, 0)),
        ],
        out_shape=[
            jax.ShapeDtypeStruct((n, c), jnp.float32),
            jax.ShapeDtypeStruct((8, c), jnp.float32),
        ],
    )(qg, p)


def _edge_norm_consts(stats, n, k):
    cnt = jnp.float32(n * k)
    sp, sp2, sps, sq, sq2 = (stats[0], stats[1], stats[2], stats[3], stats[4])
    mean = (k * sp + sq) / cnt
    e2 = (k * sp2 + 2.0 * sps + sq2) / cnt
    var = e2 - mean * mean
    inv = jax.lax.rsqrt(var + _EPS)
    z = jnp.zeros_like(mean)
    return jnp.stack([mean, inv, z, z, z, z, z, z], axis=0)


def _plain_norm_consts(stats, n):
    cnt = jnp.float32(n)
    mean = stats[0] / cnt
    var = stats[1] / cnt - mean * mean
    inv = jax.lax.rsqrt(var + _EPS)
    z = jnp.zeros_like(mean)
    return jnp.stack([mean, inv, z, z, z, z, z, z], axis=0)


def _ppf_body(gc_ref, c_ref, n_ref, p_ref):
    gx = gc_ref[:, :, 0] - c_ref[:, 0:1]
    gy = gc_ref[:, :, 1] - c_ref[:, 1:2]
    gz = gc_ref[:, :, 2] - c_ref[:, 2:3]
    nx = n_ref[:, 0:1]
    ny = n_ref[:, 1:2]
    nz = n_ref[:, 2:3]
    cx = ny * gz - nz * gy
    cy = nz * gx - nx * gz
    cz = nx * gy - ny * gx
    ss = cx * cx + cy * cy + cz * cz
    pos = (ss > 0.0).astype(jnp.float32)
    nr = jnp.sqrt(jnp.where(ss > 0.0, ss, 1.0)) * pos
    dt = nx * gx + ny * gy + nz * gz
    bz = jnp.logical_and(ss == 0.0, dt == 0.0)
    ang = jnp.arctan2(nr, jnp.where(bz, 1.0, dt))
    amax = jnp.max(ang, axis=1, keepdims=True)
    p_ref[...] = jnp.broadcast_to(amax, p_ref.shape)


def _ppf(gc3, c_pad, n_pad, *, rows):
    n = c_pad.shape[0]
    return pl.pallas_call(
        _ppf_body,
        grid=(n // rows,),
        in_specs=[
            pl.BlockSpec((rows, _PPF_K, 128), lambda i: (i, 0, 0)),
            pl.BlockSpec((rows, 16), lambda i: (i, 0)),
            pl.BlockSpec((rows, 16), lambda i: (i, 0)),
        ],
        out_specs=pl.BlockSpec((rows, 128), lambda i: (i, 0)),
        out_shape=jax.ShapeDtypeStruct((n, 128), jnp.float32),
    )(gc3, c_pad, n_pad)


def _mm2_body(x_ref, wa_ref, wb_ref, oa_ref, ob_ref):
    x = x_ref[...]
    oa_ref[...] = jnp.dot(x, wa_ref[...], preferred_element_type=jnp.float32)
    ob_ref[...] = jnp.dot(x, wb_ref[...], preferred_element_type=jnp.float32)


def _mm2(x, wa, wb, *, rows):
    n, c = x.shape
    ca = wa.shape[1]
    cb = wb.shape[1]
    return pl.pallas_call(
        _mm2_body,
        grid=(n // rows,),
        in_specs=[
            pl.BlockSpec((rows, c), lambda i: (i, 0)),
            pl.BlockSpec(wa.shape, lambda i: (0, 0)),
            pl.BlockSpec(wb.shape, lambda i: (0, 0)),
        ],
        out_specs=[
            pl.BlockSpec((rows, ca), lambda i: (i, 0)),
            pl.BlockSpec((rows, cb), lambda i: (i, 0)),
        ],
        out_shape=[
            jax.ShapeDtypeStruct((n, ca), jnp.float32),
            jax.ShapeDtypeStruct((n, cb), jnp.float32),
        ],
    )(x, wa, wb)


def _edge_mm_body(p_ref, m_ref, st_ref, wa_ref, wb_ref, f_ref, oa_ref,
                  ob_ref):
    mean = st_ref[0:1, :]
    inv = st_ref[1:2, :]
    f = _leaky((p_ref[...] + m_ref[...] - mean) * inv)
    f_ref[...] = f
    oa_ref[...] = jnp.dot(f, wa_ref[...], preferred_element_type=jnp.float32)
    ob_ref[...] = jnp.dot(f, wb_ref[...], preferred_element_type=jnp.float32)


def _edge_mm(p, m, st, wa, wb, *, rows):
    n, c = p.shape
    ca = wa.shape[1]
    cb = wb.shape[1]
    return pl.pallas_call(
        _edge_mm_body,
        grid=(n // rows,),
        in_specs=[
            pl.BlockSpec((rows, c), lambda i: (i, 0)),
            pl.BlockSpec((rows, c), lambda i: (i, 0)),
            pl.BlockSpec((8, c), lambda i: (0, 0)),
            pl.BlockSpec(wa.shape, lambda i: (0, 0)),
            pl.BlockSpec(wb.shape, lambda i: (0, 0)),
        ],
        out_specs=[
            pl.BlockSpec((rows, c), lambda i: (i, 0)),
            pl.BlockSpec((rows, ca), lambda i: (i, 0)),
            pl.BlockSpec((rows, cb), lambda i: (i, 0)),
        ],
        out_shape=[
            jax.ShapeDtypeStruct((n, c), jnp.float32),
            jax.ShapeDtypeStruct((n, ca), jnp.float32),
            jax.ShapeDtypeStruct((n, cb), jnp.float32),
        ],
    )(p, m, st, wa, wb)


def _stats_accum(s_ref, h, pid):
    @pl.when(pid == 0)
    def _():
        s_ref[...] = jnp.zeros_like(s_ref)

    s_ref[0:1, :] += jnp.sum(h, axis=0, keepdims=True)
    s_ref[1:2, :] += jnp.sum(h * h, axis=0, keepdims=True)


def _conv3_body(p2_ref, m2_ref, st2_ref, x_ref, f1_ref, w3a_ref, w3b_ref,
                w3c_ref, h3_ref, s_ref):
    mean = st2_ref[0:1, :]
    inv = st2_ref[1:2, :]
    f2 = _leaky((p2_ref[...] + m2_ref[...] - mean) * inv)
    h3 = (jnp.dot(x_ref[...], w3a_ref[...], preferred_element_type=jnp.float32)
          + jnp.dot(f1_ref[...], w3b_ref[...],
                    preferred_element_type=jnp.float32)
          + jnp.dot(f2, w3c_ref[...], preferred_element_type=jnp.float32))
    h3_ref[...] = h3
    _stats_accum(s_ref, h3, pl.program_id(0))


def _conv3(p2, m2, st2, x, f1, w3a, w3b, w3c, *, rows):
    n, c2 = p2.shape
    c = x.shape[1]
    co = w3a.shape[1]
    return pl.pallas_call(
        _conv3_body,
        grid=(n // rows,),
        in_specs=[
            pl.BlockSpec((rows, c2), lambda i: (i, 0)),
            pl.BlockSpec((rows, c2), lambda i: (i, 0)),
            pl.BlockSpec((8, c2), lambda i: (0, 0)),
            pl.BlockSpec((rows, c), lambda i: (i, 0)),
            pl.BlockSpec((rows, c), lambda i: (i, 0)),
            pl.BlockSpec(w3a.shape, lambda i: (0, 0)),
            pl.BlockSpec(w3b.shape, lambda i: (0, 0)),
            pl.BlockSpec(w3c.shape, lambda i: (0, 0)),
        ],
        out_specs=[
            pl.BlockSpec((rows, co), lambda i: (i, 0)),
            pl.BlockSpec((8, co), lambda i: (0, 0)),
        ],
        out_shape=[
            jax.ShapeDtypeStruct((n, co), jnp.float32),
            jax.ShapeDtypeStruct((8, co), jnp.float32),
        ],
    )(p2, m2, st2, x, f1, w3a, w3b, w3c)


def _fc1_body(h3_ref, st_ref, p_ref, w_ref, wp_ref, b_ref, h4_ref, s_ref):
    mean = st_ref[0:1, :]
    inv = st_ref[1:2, :]
    g = _leaky((h3_ref[...] - mean) * inv)
    h4 = (jnp.dot(g, w_ref[...], preferred_element_type=jnp.float32)
          + p_ref[:, 0:1] * wp_ref[0:1, :] + b_ref[0:1, :])
    h4_ref[...] = h4
    _stats_accum(s_ref, h4, pl.program_id(0))


def _fc1(h3, st, p, w, wp, b, *, rows):
    n, c = h3.shape
    co = w.shape[1]
    return pl.pallas_call(
        _fc1_body,
        grid=(n // rows,),
        in_specs=[
            pl.BlockSpec((rows, c), lambda i: (i, 0)),
            pl.BlockSpec((8, c), lambda i: (0, 0)),
            pl.BlockSpec((rows, 128), lambda i: (i, 0)),
            pl.BlockSpec(w.shape, lambda i: (0, 0)),
            pl.BlockSpec(wp.shape, lambda i: (0, 0)),
            pl.BlockSpec(b.shape, lambda i: (0, 0)),
        ],
        out_specs=[
            pl.BlockSpec((rows, co), lambda i: (i, 0)),
            pl.BlockSpec((8, co), lambda i: (0, 0)),
        ],
        out_shape=[
            jax.ShapeDtypeStruct((n, co), jnp.float32),
            jax.ShapeDtypeStruct((8, co), jnp.float32),
        ],
    )(h3, st, p, w, wp, b)


def _fc2_body(h4_ref, st_ref, w_ref, b_ref, h5_ref, s_ref):
    mean = st_ref[0:1, :]
    inv = st_ref[1:2, :]
    g = _leaky((h4_ref[...] - mean) * inv)
    h5 = jnp.dot(g, w_ref[...], preferred_element_type=jnp.float32) + b_ref[0:1, :]
    h5_ref[...] = h5
    _stats_accum(s_ref, h5, pl.program_id(0))


def _fc2(h4, st, w, b, *, rows):
    n, c = h4.shape
    co = w.shape[1]
    return pl.pallas_call(
        _fc2_body,
        grid=(n // rows,),
        in_specs=[
            pl.BlockSpec((rows, c), lambda i: (i, 0)),
            pl.BlockSpec((8, c), lambda i: (0, 0)),
            pl.BlockSpec(w.shape, lambda i: (0, 0)),
            pl.BlockSpec(b.shape, lambda i: (0, 0)),
        ],
        out_specs=[
            pl.BlockSpec((rows, co), lambda i: (i, 0)),
            pl.BlockSpec((8, co), lambda i: (0, 0)),
        ],
        out_shape=[
            jax.ShapeDtypeStruct((n, co), jnp.float32),
            jax.ShapeDtypeStruct((8, co), jnp.float32),
        ],
    )(h4, st, w, b)


def _final_body(h5_ref, st_ref, o_ref):
    mean = st_ref[0:1, :]
    inv = st_ref[1:2, :]
    o_ref[...] = _leaky((h5_ref[...] - mean) * inv)


def _final(h5, st, *, rows):
    n, c = h5.shape
    return pl.pallas_call(
        _final_body,
        grid=(n // rows,),
        in_specs=[
            pl.BlockSpec((rows, c), lambda i: (i, 0)),
            pl.BlockSpec((8, c), lambda i: (0, 0)),
        ],
        out_specs=pl.BlockSpec((rows, c), lambda i: (i, 0)),
        out_shape=jax.ShapeDtypeStruct((n, c), jnp.float32),
    )(h5, st)


def kernel(coords, feats, normals, W1, W2, W3, Wf1, bf1, Wf2, bf2):
    b, _, n = coords.shape
    c = feats.shape[1]
    assert b == 1
    rows = 200 if n % 200 == 0 else 8
    cw = 512 if n > 512 else 128
    nc = -(-n // cw)
    npad = nc * cw

    cT = coords[0].T
    fT = feats[0].T
    nT = normals[0].T

    c_pad8 = jnp.pad(cT, ((0, 0), (0, 5)))
    c_pad16 = jnp.pad(cT, ((0, 0), (0, 13)))
    c_pad128 = jnp.pad(cT, ((0, 0), (0, 125)))
    n_pad16 = jnp.pad(nT, ((0, 0), (0, 13)))
    call = jnp.pad(cT.T, ((0, 0), (0, npad - n)), constant_values=1e6)
    call = jnp.pad(call, ((0, 5), (0, 0)))
    call3 = call.reshape(8, nc, cw).transpose(1, 0, 2)

    knn, bq = _knn_ballquery(c_pad8, call3, n_tot=n, rows=rows, nc=nc, cw=cw)

    gc = _sc_gather(c_pad128, bq.reshape(n * _PPF_K), 128)
    p = _ppf(gc.reshape(n, _PPF_K, 128), c_pad16, n_pad16, rows=rows)

    w1d = (W1[:, :c] - W1[:, c:]).T
    w1b = W1[:, c:].T
    p1, q1 = _mm2(fT, w1d, w1b, rows=rows)
    qg1 = _sc_gather(q1, knn.reshape(n * _GCN_K), 128)
    m1, st1 = _nbr_reduce(qg1.reshape(n, _GCN_K, c), p1, rows=rows)
    ncs1 = _edge_norm_consts(st1, n, _GCN_K)

    w2d = (W2[:, :c] - W2[:, c:]).T
    w2b = W2[:, c:].T
    f1, p2, q2 = _edge_mm(p1, m1, ncs1, w2d, w2b, rows=rows)
    knn_flat = knn.reshape(n * _GCN_K)
    qg2a = _sc_gather(q2[:, :c], knn_flat, 128)
    qg2b = _sc_gather(q2[:, c:], knn_flat, 128)
    m2, st2 = _nbr_reduce2(qg2a.reshape(n, _GCN_K, c),
                           qg2b.reshape(n, _GCN_K, c), p2, rows=rows)
    ncs2 = _edge_norm_consts(st2, n, _GCN_K)

    w3a = W3[:, :c].T
    w3b = W3[:, c:2 * c].T
    w3c = W3[:, 2 * c:].T
    h3, st3 = _conv3(p2, m2, ncs2, fT, f1, w3a, w3b, w3c, rows=rows)
    ncs3 = _plain_norm_consts(st3, n)

    wp = Wf1[:, 0:1].T
    wf1g = Wf1[:, 1:].T
    bf1r = bf1.reshape(1, -1)
    h4, st4 = _fc1(h3, ncs3, p, wf1g, wp, bf1r, rows=rows)
    ncs4 = _plain_norm_consts(st4, n)

    h5, st5 = _fc2(h4, ncs4, Wf2.T, bf2.reshape(1, -1), rows=rows)
    ncs5 = _plain_norm_consts(st5, n)

    out = _final(h5, ncs5, rows=rows)
    return out.T[None]


# speedup vs baseline: 3.9160x; 1.0001x over previous
import functools

import jax
import jax.numpy as jnp
from jax.experimental import pallas as pl
from jax.experimental.pallas import tpu as pltpu
from jax.experimental.pallas import tpu_sc as plsc

_GCN_K = 16
_PPF_K = 32
_R2 = 0.3 ** 2
_EPS = 1e-5


def _leaky(x):
    return jnp.where(x >= 0.0, x, 0.2 * x)


def _knn_bq_body(crow_ref, call_ref, knn_ref, bq_ref,
                 sq_ref, hit_ref, kn_s, bq_s,
                 *, n_tot, nc, cw, rows):
    f32 = jnp.float32
    i32 = jnp.int32
    bigi = i32(nc * cw + 1)
    ax = crow_ref[:, 0:1]
    ay = crow_ref[:, 1:2]
    az = crow_ref[:, 2:3]
    aa = ax * ax + ay * ay + az * az
    rx = ax.astype(jnp.bfloat16).astype(jnp.float32)
    ry = ay.astype(jnp.bfloat16).astype(jnp.float32)
    rz = az.astype(jnp.bfloat16).astype(jnp.float32)

    def _fold(x, acc):
        return jnp.minimum(acc, x)

    def _lanemin(acc):
        return jnp.min(acc, axis=-1, keepdims=True)

    def fill_body(j, macc):
        bx = call_ref[j, 0:1, :]
        by = call_ref[j, 1:2, :]
        bz = call_ref[j, 2:3, :]
        sx = bx.astype(jnp.bfloat16).astype(jnp.float32)
        sy = by.astype(jnp.bfloat16).astype(jnp.float32)
        sz = bz.astype(jnp.bfloat16).astype(jnp.float32)
        ab = rx * sx + ry * sy + rz * sz
        bb = bx * bx + by * by + bz * bz
        sq = aa - 2.0 * ab + bb
        sq_ref[j] = sq
        col = j * cw + jax.lax.broadcasted_iota(i32, (rows, cw), 1)
        hit_ref[j] = jnp.where(sq < f32(_R2), col, n_tot)
        return _fold(sq, macc)

    m0 = _lanemin(jax.lax.fori_loop(0, nc, fill_body,
                                    jnp.full((rows, cw), jnp.inf, f32)))

    def bq_body(t, lastc):
        def mn(j, acc):
            h = hit_ref[j]
            return _fold(jnp.where(h > lastc, h, n_tot), acc)
        m = _lanemin(jax.lax.fori_loop(0, nc, mn,
                                       jnp.full((rows, cw), n_tot, i32)))
        bq_s[t] = m
        return m

    jax.lax.fori_loop(0, _PPF_K, bq_body,
                      jnp.full((rows, 1), -1, i32))

    def ix_pass(m):
        def ix(j, acc):
            ch = sq_ref[j]
            col = j * cw + jax.lax.broadcasted_iota(i32, (rows, cw), 1)
            return _fold(jnp.where(ch == m, col, bigi), acc)
        return _lanemin(jax.lax.fori_loop(0, nc, ix,
                                          jnp.full((rows, cw), bigi, i32)))

    def kn_body(t, m):
        ii = ix_pass(m)
        kn_s[t] = ii

        def mskmin(j, acc):
            col = j * cw + jax.lax.broadcasted_iota(i32, (rows, cw), 1)
            v = jnp.where(col == ii, jnp.inf, sq_ref[j])
            sq_ref[j] = v
            return _fold(v, acc)

        return _lanemin(jax.lax.fori_loop(0, nc, mskmin,
                                          jnp.full((rows, cw), jnp.inf, f32)))

    m_last = jax.lax.fori_loop(0, _GCN_K, kn_body, m0)
    kn_s[_GCN_K] = ix_pass(m_last)

    first = bq_s[0]
    for s in range(_PPF_K):
        v = bq_s[s]
        bq_ref[:, s:s + 1] = jnp.where(v == n_tot, first, v)
    for t in range(1, _GCN_K + 1):
        knn_ref[:, t - 1:t] = kn_s[t]


def _knn_ballquery(c_pad8, call3, *, n_tot, rows, nc, cw):
    nblk = n_tot // rows
    body = functools.partial(_knn_bq_body, n_tot=n_tot, nc=nc, cw=cw,
                             rows=rows)
    return pl.pallas_call(
        body,
        grid=(nblk,),
        in_specs=[
            pl.BlockSpec((rows, 8), lambda i: (i, 0)),
            pl.BlockSpec((nc, 8, cw), lambda i: (0, 0, 0)),
        ],
        out_specs=[
            pl.BlockSpec((rows, _GCN_K), lambda i: (i, 0)),
            pl.BlockSpec((rows, _PPF_K), lambda i: (i, 0)),
        ],
        out_shape=[
            jax.ShapeDtypeStruct((n_tot, _GCN_K), jnp.int32),
            jax.ShapeDtypeStruct((n_tot, _PPF_K), jnp.int32),
        ],
        scratch_shapes=[
            pltpu.VMEM((nc, rows, cw), jnp.float32),
            pltpu.VMEM((nc, rows, cw), jnp.int32),
            pltpu.VMEM((_GCN_K + 1, rows, 1), jnp.int32),
            pltpu.VMEM((_PPF_K, rows, 1), jnp.int32),
        ],
        compiler_params=pltpu.CompilerParams(
            dimension_semantics=("parallel",)),
    )(c_pad8, call3)


def _sc_gather(data, idx_flat, window):
    ni = idx_flat.shape[0]
    d = data.shape[1]
    assert ni % window == 0
    mesh = plsc.VectorSubcoreMesh(core_axis_name="c", subcore_axis_name="s")
    idx2 = idx_flat.reshape(1, ni)

    @functools.partial(
        pl.kernel,
        out_type=jax.ShapeDtypeStruct((ni, d), data.dtype),
        mesh=mesh,
    )
    def gather_kernel(x_hbm, i_hbm, o_hbm):
        def body(i_vmem, o_vmem):
            pltpu.sync_copy(x_hbm.at[i_vmem.at[0]], o_vmem)

        pltpu.emit_pipeline(
            body,
            grid=(ni // window,),
            in_specs=[pl.BlockSpec((1, window), index_map=lambda i: (0, i))],
            out_specs=[pl.BlockSpec((window, d), index_map=lambda i: (i, 0))],
            core_axis_name=("c", "s"),
            dimension_semantics=(pltpu.PARALLEL,),
        )(i_hbm, o_hbm)

    return gather_kernel(data, idx2)


def _nbr_reduce_body(qg_ref, p_ref, m_ref, s_ref):
    qg = qg_ref[...]
    mx = jnp.max(qg, axis=1)
    sm = jnp.sum(qg, axis=1)
    s2 = jnp.sum(qg * qg, axis=1)
    m_ref[...] = mx
    p = p_ref[...]

    @pl.when(pl.program_id(0) == 0)
    def _():
        s_ref[...] = jnp.zeros_like(s_ref)

    s_ref[0:1, :] += jnp.sum(p, axis=0, keepdims=True)
    s_ref[1:2, :] += jnp.sum(p * p, axis=0, keepdims=True)
    s_ref[2:3, :] += jnp.sum(p * sm, axis=0, keepdims=True)
    s_ref[3:4, :] += jnp.sum(sm, axis=0, keepdims=True)
    s_ref[4:5, :] += jnp.sum(s2, axis=0, keepdims=True)


def _nbr_reduce2_body(qa_ref, qb_ref, p_ref, m_ref, s_ref):
    c = qa_ref.shape[2]
    p = p_ref[...]

    @pl.when(pl.program_id(0) == 0)
    def _():
        s_ref[...] = jnp.zeros_like(s_ref)

    for h, qref in ((0, qa_ref), (1, qb_ref)):
        lo, hi = h * c, (h + 1) * c
        qg = qref[...]
        mx = jnp.max(qg, axis=1)
        sm = jnp.sum(qg, axis=1)
        s2 = jnp.sum(qg * qg, axis=1)
        m_ref[:, lo:hi] = mx
        ph = p[:, lo:hi]
        s_ref[0:1, lo:hi] += jnp.sum(ph, axis=0, keepdims=True)
        s_ref[1:2, lo:hi] += jnp.sum(ph * ph, axis=0, keepdims=True)
        s_ref[2:3, lo:hi] += jnp.sum(ph * sm, axis=0, keepdims=True)
        s_ref[3:4, lo:hi] += jnp.sum(sm, axis=0, keepdims=True)
        s_ref[4:5, lo:hi] += jnp.sum(s2, axis=0, keepdims=True)


def _nbr_reduce2(qa, qb, p, *, rows):
    n, k, c = qa.shape
    return pl.pallas_call(
        _nbr_reduce2_body,
        grid=(n // rows,),
        in_specs=[
            pl.BlockSpec((rows, k, c), lambda i: (i, 0, 0)),
            pl.BlockSpec((rows, k, c), lambda i: (i, 0, 0)),
            pl.BlockSpec((rows, 2 * c), lambda i: (i, 0)),
        ],
        out_specs=[
            pl.BlockSpec((rows, 2 * c), lambda i: (i, 0)),
            pl.BlockSpec((8, 2 * c), lambda i: (0, 0)),
        ],
        out_shape=[
            jax.ShapeDtypeStruct((n, 2 * c), jnp.float32),
            jax.ShapeDtypeStruct((8, 2 * c), jnp.float32),
        ],
    )(qa, qb, p)


def _nbr_reduce(qg, p, *, rows):
    n, k, c = qg.shape
    return pl.pallas_call(
        _nbr_reduce_body,
        grid=(n // rows,),
        in_specs=[
            pl.BlockSpec((rows, k, c), lambda i: (i, 0, 0)),
            pl.BlockSpec((rows, c), lambda i: (i, 0)),
        ],
        out_specs=[
            pl.BlockSpec((rows, c), lambda i: (i, 0)),
            pl.BlockSpec((8, c), lambda i: (0, 0)),
        ],
        out_shape=[
            jax.ShapeDtypeStruct((n, c), jnp.float32),
            jax.ShapeDtypeStruct((8, c), jnp.float32),
        ],
    )(qg, p)


def _edge_norm_consts(stats, n, k):
    cnt = jnp.float32(n * k)
    sp, sp2, sps, sq, sq2 = (stats[0], stats[1], stats[2], stats[3], stats[4])
    mean = (k * sp + sq) / cnt
    e2 = (k * sp2 + 2.0 * sps + sq2) / cnt
    var = e2 - mean * mean
    inv = jax.lax.rsqrt(var + _EPS)
    z = jnp.zeros_like(mean)
    return jnp.stack([mean, inv, z, z, z, z, z, z], axis=0)


def _plain_norm_consts(stats, n):
    cnt = jnp.float32(n)
    mean = stats[0] / cnt
    var = stats[1] / cnt - mean * mean
    inv = jax.lax.rsqrt(var + _EPS)
    z = jnp.zeros_like(mean)
    return jnp.stack([mean, inv, z, z, z, z, z, z], axis=0)


def _ppf_body(gc_ref, c_ref, n_ref, p_ref):
    gx = gc_ref[:, :, 0] - c_ref[:, 0:1]
    gy = gc_ref[:, :, 1] - c_ref[:, 1:2]
    gz = gc_ref[:, :, 2] - c_ref[:, 2:3]
    nx = n_ref[:, 0:1]
    ny = n_ref[:, 1:2]
    nz = n_ref[:, 2:3]
    cx = ny * gz - nz * gy
    cy = nz * gx - nx * gz
    cz = nx * gy - ny * gx
    ss = cx * cx + cy * cy + cz * cz
    pos = (ss > 0.0).astype(jnp.float32)
    nr = jnp.sqrt(jnp.where(ss > 0.0, ss, 1.0)) * pos
    dt = nx * gx + ny * gy + nz * gz
    bz = jnp.logical_and(ss == 0.0, dt == 0.0)
    ang = jnp.arctan2(nr, jnp.where(bz, 1.0, dt))
    amax = jnp.max(ang, axis=1, keepdims=True)
    p_ref[...] = jnp.broadcast_to(amax, p_ref.shape)


def _ppf(gc3, c_pad, n_pad, *, rows):
    n = c_pad.shape[0]
    return pl.pallas_call(
        _ppf_body,
        grid=(n // rows,),
        in_specs=[
            pl.BlockSpec((rows, _PPF_K, 128), lambda i: (i, 0, 0)),
            pl.BlockSpec((rows, 16), lambda i: (i, 0)),
            pl.BlockSpec((rows, 16), lambda i: (i, 0)),
        ],
        out_specs=pl.BlockSpec((rows, 128), lambda i: (i, 0)),
        out_shape=jax.ShapeDtypeStruct((n, 128), jnp.float32),
    )(gc3, c_pad, n_pad)


def _mm2_body(x_ref, wa_ref, wb_ref, oa_ref, ob_ref):
    x = x_ref[...]
    oa_ref[...] = jnp.dot(x, wa_ref[...], preferred_element_type=jnp.float32)
    ob_ref[...] = jnp.dot(x, wb_ref[...], preferred_element_type=jnp.float32)


def _mm2(x, wa, wb, *, rows):
    n, c = x.shape
    ca = wa.shape[1]
    cb = wb.shape[1]
    return pl.pallas_call(
        _mm2_body,
        grid=(n // rows,),
        in_specs=[
            pl.BlockSpec((rows, c), lambda i: (i, 0)),
            pl.BlockSpec(wa.shape, lambda i: (0, 0)),
            pl.BlockSpec(wb.shape, lambda i: (0, 0)),
        ],
        out_specs=[
            pl.BlockSpec((rows, ca), lambda i: (i, 0)),
            pl.BlockSpec((rows, cb), lambda i: (i, 0)),
        ],
        out_shape=[
            jax.ShapeDtypeStruct((n, ca), jnp.float32),
            jax.ShapeDtypeStruct((n, cb), jnp.float32),
        ],
    )(x, wa, wb)


def _edge_mm_body(p_ref, m_ref, st_ref, wa_ref, wb_ref, f_ref, oa_ref,
                  ob_ref):
    mean = st_ref[0:1, :]
    inv = st_ref[1:2, :]
    f = _leaky((p_ref[...] + m_ref[...] - mean) * inv)
    f_ref[...] = f
    oa_ref[...] = jnp.dot(f, wa_ref[...], preferred_element_type=jnp.float32)
    ob_ref[...] = jnp.dot(f, wb_ref[...], preferred_element_type=jnp.float32)


def _edge_mm(p, m, st, wa, wb, *, rows):
    n, c = p.shape
    ca = wa.shape[1]
    cb = wb.shape[1]
    return pl.pallas_call(
        _edge_mm_body,
        grid=(n // rows,),
        in_specs=[
            pl.BlockSpec((rows, c), lambda i: (i, 0)),
            pl.BlockSpec((rows, c), lambda i: (i, 0)),
            pl.BlockSpec((8, c), lambda i: (0, 0)),
            pl.BlockSpec(wa.shape, lambda i: (0, 0)),
            pl.BlockSpec(wb.shape, lambda i: (0, 0)),
        ],
        out_specs=[
            pl.BlockSpec((rows, c), lambda i: (i, 0)),
            pl.BlockSpec((rows, ca), lambda i: (i, 0)),
            pl.BlockSpec((rows, cb), lambda i: (i, 0)),
        ],
        out_shape=[
            jax.ShapeDtypeStruct((n, c), jnp.float32),
            jax.ShapeDtypeStruct((n, ca), jnp.float32),
            jax.ShapeDtypeStruct((n, cb), jnp.float32),
        ],
    )(p, m, st, wa, wb)


def _stats_accum(s_ref, h, pid):
    @pl.when(pid == 0)
    def _():
        s_ref[...] = jnp.zeros_like(s_ref)

    s_ref[0:1, :] += jnp.sum(h, axis=0, keepdims=True)
    s_ref[1:2, :] += jnp.sum(h * h, axis=0, keepdims=True)


def _conv3_body(p2_ref, m2_ref, st2_ref, x_ref, f1_ref, w3a_ref, w3b_ref,
                w3c_ref, h3_ref, s_ref):
    mean = st2_ref[0:1, :]
    inv = st2_ref[1:2, :]
    f2 = _leaky((p2_ref[...] + m2_ref[...] - mean) * inv)
    h3 = (jnp.dot(x_ref[...], w3a_ref[...], preferred_element_type=jnp.float32)
          + jnp.dot(f1_ref[...], w3b_ref[...],
                    preferred_element_type=jnp.float32)
          + jnp.dot(f2, w3c_ref[...], preferred_element_type=jnp.float32))
    h3_ref[...] = h3
    _stats_accum(s_ref, h3, pl.program_id(0))


def _conv3(p2, m2, st2, x, f1, w3a, w3b, w3c, *, rows):
    n, c2 = p2.shape
    c = x.shape[1]
    co = w3a.shape[1]
    return pl.pallas_call(
        _conv3_body,
        grid=(n // rows,),
        in_specs=[
            pl.BlockSpec((rows, c2), lambda i: (i, 0)),
            pl.BlockSpec((rows, c2), lambda i: (i, 0)),
            pl.BlockSpec((8, c2), lambda i: (0, 0)),
            pl.BlockSpec((rows, c), lambda i: (i, 0)),
            pl.BlockSpec((rows, c), lambda i: (i, 0)),
            pl.BlockSpec(w3a.shape, lambda i: (0, 0)),
            pl.BlockSpec(w3b.shape, lambda i: (0, 0)),
            pl.BlockSpec(w3c.shape, lambda i: (0, 0)),
        ],
        out_specs=[
            pl.BlockSpec((rows, co), lambda i: (i, 0)),
            pl.BlockSpec((8, co), lambda i: (0, 0)),
        ],
        out_shape=[
            jax.ShapeDtypeStruct((n, co), jnp.float32),
            jax.ShapeDtypeStruct((8, co), jnp.float32),
        ],
    )(p2, m2, st2, x, f1, w3a, w3b, w3c)


def _fc1_body(h3_ref, st_ref, p_ref, w_ref, wp_ref, b_ref, h4_ref, s_ref):
    mean = st_ref[0:1, :]
    inv = st_ref[1:2, :]
    g = _leaky((h3_ref[...] - mean) * inv)
    h4 = (jnp.dot(g, w_ref[...], preferred_element_type=jnp.float32)
          + p_ref[:, 0:1] * wp_ref[0:1, :] + b_ref[0:1, :])
    h4_ref[...] = h4
    _stats_accum(s_ref, h4, pl.program_id(0))


def _fc1(h3, st, p, w, wp, b, *, rows):
    n, c = h3.shape
    co = w.shape[1]
    return pl.pallas_call(
        _fc1_body,
        grid=(n // rows,),
        in_specs=[
            pl.BlockSpec((rows, c), lambda i: (i, 0)),
            pl.BlockSpec((8, c), lambda i: (0, 0)),
            pl.BlockSpec((rows, 128), lambda i: (i, 0)),
            pl.BlockSpec(w.shape, lambda i: (0, 0)),
            pl.BlockSpec(wp.shape, lambda i: (0, 0)),
            pl.BlockSpec(b.shape, lambda i: (0, 0)),
        ],
        out_specs=[
            pl.BlockSpec((rows, co), lambda i: (i, 0)),
            pl.BlockSpec((8, co), lambda i: (0, 0)),
        ],
        out_shape=[
            jax.ShapeDtypeStruct((n, co), jnp.float32),
            jax.ShapeDtypeStruct((8, co), jnp.float32),
        ],
    )(h3, st, p, w, wp, b)


def _fc2_body(h4_ref, st_ref, w_ref, b_ref, h5_ref, s_ref):
    mean = st_ref[0:1, :]
    inv = st_ref[1:2, :]
    g = _leaky((h4_ref[...] - mean) * inv)
    h5 = jnp.dot(g, w_ref[...], preferred_element_type=jnp.float32) + b_ref[0:1, :]
    h5_ref[...] = h5
    _stats_accum(s_ref, h5, pl.program_id(0))


def _fc2(h4, st, w, b, *, rows):
    n, c = h4.shape
    co = w.shape[1]
    return pl.pallas_call(
        _fc2_body,
        grid=(n // rows,),
        in_specs=[
            pl.BlockSpec((rows, c), lambda i: (i, 0)),
            pl.BlockSpec((8, c), lambda i: (0, 0)),
            pl.BlockSpec(w.shape, lambda i: (0, 0)),
            pl.BlockSpec(b.shape, lambda i: (0, 0)),
        ],
        out_specs=[
            pl.BlockSpec((rows, co), lambda i: (i, 0)),
            pl.BlockSpec((8, co), lambda i: (0, 0)),
        ],
        out_shape=[
            jax.ShapeDtypeStruct((n, co), jnp.float32),
            jax.ShapeDtypeStruct((8, co), jnp.float32),
        ],
    )(h4, st, w, b)


def _final_body(h5_ref, st_ref, o_ref):
    mean = st_ref[0:1, :]
    inv = st_ref[1:2, :]
    o_ref[...] = _leaky((h5_ref[...] - mean) * inv)


def _final(h5, st, *, rows):
    n, c = h5.shape
    return pl.pallas_call(
        _final_body,
        grid=(n // rows,),
        in_specs=[
            pl.BlockSpec((rows, c), lambda i: (i, 0)),
            pl.BlockSpec((8, c), lambda i: (0, 0)),
        ],
        out_specs=pl.BlockSpec((rows, c), lambda i: (i, 0)),
        out_shape=jax.ShapeDtypeStruct((n, c), jnp.float32),
    )(h5, st)


def kernel(coords, feats, normals, W1, W2, W3, Wf1, bf1, Wf2, bf2):
    b, _, n = coords.shape
    c = feats.shape[1]
    assert b == 1
    rows = 200 if n % 200 == 0 else 8
    cw = 512 if n > 512 else 128
    nc = -(-n // cw)
    npad = nc * cw

    cT = coords[0].T
    fT = feats[0].T
    nT = normals[0].T

    c_pad8 = jnp.pad(cT, ((0, 0), (0, 5)))
    c_pad16 = jnp.pad(cT, ((0, 0), (0, 13)))
    c_pad128 = jnp.pad(cT, ((0, 0), (0, 125)))
    n_pad16 = jnp.pad(nT, ((0, 0), (0, 13)))
    call = jnp.pad(cT.T, ((0, 0), (0, npad - n)), constant_values=1e6)
    call = jnp.pad(call, ((0, 5), (0, 0)))
    call3 = call.reshape(8, nc, cw).transpose(1, 0, 2)

    knn, bq = _knn_ballquery(c_pad8, call3, n_tot=n, rows=rows, nc=nc, cw=cw)

    gc = _sc_gather(c_pad128, bq.reshape(n * _PPF_K), 128)
    p = _ppf(gc.reshape(n, _PPF_K, 128), c_pad16, n_pad16, rows=rows)

    w1d = (W1[:, :c] - W1[:, c:]).T
    w1b = W1[:, c:].T
    p1, q1 = _mm2(fT, w1d, w1b, rows=rows)
    qg1 = _sc_gather(q1, knn.reshape(n * _GCN_K), 128)
    m1, st1 = _nbr_reduce(qg1.reshape(n, _GCN_K, c), p1, rows=rows)
    ncs1 = _edge_norm_consts(st1, n, _GCN_K)

    w2d = (W2[:, :c] - W2[:, c:]).T
    w2b = W2[:, c:].T
    f1, p2, q2 = _edge_mm(p1, m1, ncs1, w2d, w2b, rows=rows)
    knn_flat = knn.reshape(n * _GCN_K)
    qg2a = _sc_gather(q2[:, :c], knn_flat, 128)
    qg2b = _sc_gather(q2[:, c:], knn_flat, 128)
    m2, st2 = _nbr_reduce2(qg2a.reshape(n, _GCN_K, c),
                           qg2b.reshape(n, _GCN_K, c), p2, rows=rows)
    ncs2 = _edge_norm_consts(st2, n, _GCN_K)

    w3a = W3[:, :c].T
    w3b = W3[:, c:2 * c].T
    w3c = W3[:, 2 * c:].T
    h3, st3 = _conv3(p2, m2, ncs2, fT, f1, w3a, w3b, w3c, rows=rows)
    ncs3 = _plain_norm_consts(st3, n)

    wp = Wf1[:, 0:1].T
    wf1g = Wf1[:, 1:].T
    bf1r = bf1.reshape(1, -1)
    h4, st4 = _fc1(h3, ncs3, p, wf1g, wp, bf1r, rows=rows)
    ncs4 = _plain_norm_consts(st4, n)

    h5, st5 = _fc2(h4, ncs4, Wf2.T, bf2.reshape(1, -1), rows=rows)
    ncs5 = _plain_norm_consts(st5, n)

    out = _final(h5, ncs5, rows=rows)
    return out.T[None]
